# baseline (speedup 1.0000x reference)
import functools

import numpy as np
import jax
import jax.numpy as jnp
from jax import lax
from jax.experimental import pallas as pl
from jax.experimental.pallas import tpu as pltpu
from jax.experimental.pallas import tpu_sc as plsc

_NUM_CORES = 2
_NUM_SUBCORES = 16
_LANES = 16
_NW = _NUM_CORES * _NUM_SUBCORES


def _u_thresholds(i_dim: int, m_dim: int) -> jax.Array:
    u = jax.random.uniform(jax.random.key(42), (i_dim, m_dim, 1),
                           dtype=jnp.float32)
    return u.reshape(i_dim * m_dim)


@functools.lru_cache(maxsize=2)
def _build_sc_call(rows: int, np1: int):
    rows_per_w = rows // _NW
    chunk = 512
    assert rows_per_w % chunk == 0 and chunk % _LANES == 0
    n_chunks = rows_per_w // chunk
    chunk_words = chunk * np1

    mesh = plsc.VectorSubcoreMesh(core_axis_name="c", subcore_axis_name="s")

    @functools.partial(
        pl.kernel,
        out_type=jax.ShapeDtypeStruct((rows,), jnp.int32),
        mesh=mesh,
        compiler_params=pltpu.CompilerParams(needs_layout_passes=False),
        scratch_types=[
            pltpu.VMEM((chunk_words,), jnp.float32),
            pltpu.VMEM((rows_per_w,), jnp.float32),
            pltpu.VMEM((rows_per_w,), jnp.int32),
        ],
    )
    def sc_count(pi_hbm, u_hbm, out_hbm, pi_v, u_v, out_v):
        wid = lax.axis_index("s") * _NUM_CORES + lax.axis_index("c")
        row0 = wid * rows_per_w
        pltpu.sync_copy(u_hbm.at[pl.ds(row0, rows_per_w)], u_v)

        lane = lax.iota(jnp.int32, _LANES)

        @pl.loop(0, n_chunks)
        def _chunk_loop(ci):
            pltpu.sync_copy(
                pi_hbm.at[pl.ds((row0 + ci * chunk) * np1, chunk_words)],
                pi_v)

            @pl.loop(0, chunk // _LANES)
            def _group_loop(g):
                out_base = ci * chunk + g * _LANES
                idx = (g * _LANES + lane) * np1
                u_vec = u_v[pl.ds(out_base, _LANES)]
                acc = jnp.zeros((_LANES,), jnp.float32)
                cnt = jnp.zeros((_LANES,), jnp.int32)
                for k in range(np1):
                    v = plsc.load_gather(pi_v, [idx + k])
                    acc = acc + v
                    cnt = jnp.where(u_vec > acc, cnt + 1, cnt)
                out_v[pl.ds(out_base, _LANES)] = cnt

        pltpu.sync_copy(out_v, out_hbm.at[pl.ds(row0, rows_per_w)])

    return sc_count


def kernel(pi_vectors):
    i_dim, m_dim, np1 = pi_vectors.shape
    rows = i_dim * m_dim
    u = _u_thresholds(i_dim, m_dim)
    pi_flat = pi_vectors.reshape(rows * np1)
    out = _build_sc_call(rows, np1)(pi_flat, u)
    return out.reshape(i_dim, m_dim)
